# baseline (speedup 1.0000x reference)
import functools

import jax
import jax.numpy as jnp
from jax import lax
from jax.experimental import pallas as pl
from jax.experimental.pallas import tpu as pltpu
from jax.experimental.pallas import tpu_sc as plsc

_B = 8
_N = 4096
_D = 64
_NID = 32
_TPB = 4
_NPT = _N // _TPB
_DELTA_V = 0.5
_TWO_DELTA_D = 3.0
_GAMMA = 0.001


def _sqrt16(v):
    i = plsc.bitcast(v, jnp.int32)
    g = plsc.bitcast(lax.shift_right_logical(i, 1) + 0x1FBD1DF5, jnp.float32)
    for _ in range(4):
        g = 0.5 * (g + v / g)
    return g


_mesh = plsc.VectorSubcoreMesh(
    core_axis_name="c", subcore_axis_name="s", num_cores=2, num_subcores=16
)


@functools.partial(
    pl.kernel,
    out_type=jax.ShapeDtypeStruct((_B, 16), jnp.float32),
    mesh=_mesh,
    compiler_params=pltpu.CompilerParams(needs_layout_passes=False),
    scratch_types=[
        pltpu.VMEM((_NPT * _D,), jnp.float32),
        pltpu.VMEM((_NPT,), jnp.int32),
        pltpu.VMEM((_NID * _D,), jnp.float32),
        pltpu.VMEM((_NID * _D,), jnp.float32),
        pltpu.VMEM((_NID * 16,), jnp.float32),
        pltpu.VMEM((_NID * 16,), jnp.float32),
        pltpu.VMEM((_NID * 16,), jnp.float32),
        pltpu.VMEM((16 * 16,), jnp.float32),
        pltpu.VMEM((_D * 16,), jnp.float32),
        pltpu.VMEM((16,), jnp.float32),
        pltpu.VMEM_SHARED((16, _NID * _D), jnp.float32),
        pltpu.VMEM_SHARED((16, _NID * 16), jnp.float32),
        pltpu.VMEM_SHARED((16, _NID * 16), jnp.float32),
    ],
)
def _loss_kernel(
    emb_hbm, lab_hbm, out_hbm,
    emb_v, lab_v, sums_v, centers_v, counts_tab, hinge_tab, rbuf,
    qtab, ct_v, obuf, sums_sh, counts_sh, hinge_sh,
):
    c = lax.axis_index("c")
    s = lax.axis_index("s")
    batch = c * 4 + lax.shift_right_logical(s, 2)
    qtr = lax.bitwise_and(s, _TPB - 1)
    s0 = s - qtr
    zero16 = jnp.zeros((16,), jnp.float32)
    iota16 = lax.iota(jnp.int32, 16)
    ids16 = iota16 + 1
    lane0 = jnp.where(iota16 == 0, jnp.float32(1.0), jnp.float32(0.0))
    _onehots = [
        jnp.where(iota16 == k, jnp.float32(1.0), jnp.float32(0.0))
        for k in range(16)
    ]

    def _hsum(v):
        r = v
        for sh in (8, 4, 2, 1):
            obuf[...] = r
            r = r + plsc.load_gather(obuf, [jnp.bitwise_xor(iota16, sh)])
        return r

    def _colsum(tab, row0):
        acc = None
        for j in range(16):
            colv = plsc.load_gather(tab, [(iota16 + row0) * 16 + j])
            acc = colv if acc is None else acc + colv
        return acc

    pltpu.sync_copy(emb_hbm.at[batch, pl.ds(qtr * (_NPT * _D), _NPT * _D)], emb_v)
    pltpu.sync_copy(lab_hbm.at[batch, pl.ds(qtr * _NPT, _NPT)], lab_v)

    def _zb(i, carry):
        sums_v[pl.ds(i * 16, 16)] = zero16
        return carry

    lax.fori_loop(0, (_NID * _D) // 16, _zb, 0)

    def _zt(i, carry):
        counts_tab[pl.ds(i * 16, 16)] = zero16
        hinge_tab[pl.ds(i * 16, 16)] = zero16
        return carry

    lax.fori_loop(0, _NID, _zt, 0)

    def _p1(g, carry):
        lvec = lab_v[pl.ds(g * 16, 16)]
        base = g * 16
        for k in range(16):
            l = lvec[k]
            co = l * _D
            eo = (base + k) * _D
            for kk in range(_D // 16):
                sl = pl.ds(co + kk * 16, 16)
                sums_v[sl] = sums_v[sl] + emb_v[pl.ds(eo + kk * 16, 16)]
            cl = pl.ds(l * 16, 16)
            counts_tab[cl] = counts_tab[cl] + lane0
        return carry

    lax.fori_loop(0, _NPT // 16, _p1, 0)

    pltpu.sync_copy(sums_v, sums_sh.at[s])
    pltpu.sync_copy(counts_tab, counts_sh.at[s])
    plsc.subcore_barrier()

    @pl.when(qtr == 0)
    def _lead_centers():
        for j in range(1, _TPB):
            pltpu.sync_copy(counts_sh.at[s + j], rbuf)

            def _accc(i, carry):
                sl = pl.ds(i * 16, 16)
                counts_tab[sl] = counts_tab[sl] + rbuf[sl]
                return carry

            lax.fori_loop(0, _NID, _accc, 0)
            pltpu.sync_copy(sums_sh.at[s + j], centers_v)

            def _accs(i, carry):
                sl = pl.ds(i * 16, 16)
                sums_v[sl] = sums_v[sl] + centers_v[sl]
                return carry

            lax.fori_loop(0, (_NID * _D) // 16, _accs, 0)

        rv0 = 1.0 / jnp.maximum(_colsum(counts_tab, 0), 1.0)
        rv1 = 1.0 / jnp.maximum(_colsum(counts_tab, 16), 1.0)
        for i in range(_NID):
            r = rv0[i] if i < 16 else rv1[i - 16]
            for kk in range(_D // 16):
                sl = pl.ds(i * _D + kk * 16, 16)
                centers_v[sl] = sums_v[sl] * r
        pltpu.sync_copy(centers_v, sums_sh.at[s])
        pltpu.sync_copy(counts_tab, counts_sh.at[s])

    plsc.subcore_barrier()
    pltpu.sync_copy(sums_sh.at[s0], centers_v)
    pltpu.sync_copy(counts_sh.at[s0], counts_tab)

    def _p2(g, carry):
        base = g * 16
        lvec = lab_v[pl.ds(base, 16)]
        for k in range(16):
            l = lvec[k]
            co = l * _D
            eo = (base + k) * _D
            acc = None
            for kk in range(_D // 16):
                e = emb_v[pl.ds(eo + kk * 16, 16)]
                cc = centers_v[pl.ds(co + kk * 16, 16)]
                d = e - cc
                t = d * d
                acc = t if acc is None else acc + t
            qtab[pl.ds(k * 16, 16)] = acc
        qv = _colsum(qtab, 0)
        hv = jnp.maximum(_sqrt16(qv) - _DELTA_V, 0.0)
        for k in range(16):
            l = lvec[k]
            hl = pl.ds(l * 16, 16)
            hinge_tab[hl] = hinge_tab[hl] + hv * _onehots[k]
        return carry

    lax.fori_loop(0, _NPT // 16, _p2, 0)

    pltpu.sync_copy(hinge_tab, hinge_sh.at[s])
    plsc.subcore_barrier()

    @pl.when(qtr == 0)
    def _lead_final():
        for j in range(1, _TPB):
            pltpu.sync_copy(hinge_sh.at[s + j], rbuf)

            def _acch(i, carry):
                sl = pl.ds(i * 16, 16)
                hinge_tab[sl] = hinge_tab[sl] + rbuf[sl]
                return carry

            lax.fori_loop(0, _NID, _acch, 0)

        cnt = _colsum(counts_tab, 1)
        hin = _colsum(hinge_tab, 1)
        presentf = jnp.where(cnt > 0.0, 1.0, 0.0)
        n_inst_v = _hsum(presentf)
        safe_n_v = jnp.maximum(n_inst_v, 1.0)
        mean_d = hin / jnp.maximum(cnt, 1.0)
        var_term_v = _hsum(mean_d * presentf) / safe_n_v

        def _tr(dd, carry):
            col = plsc.load_gather(centers_v, [ids16 * _D + dd])
            ct_v[pl.ds(dd * 16, 16)] = col
            return carry + col * col

        norm2 = lax.fori_loop(0, _D, _tr, zero16)
        reg_term_v = _hsum(_sqrt16(norm2) * presentf) / safe_n_v

        pairsum_v = zero16
        for i in range(1, 17):
            def _dot(dd, carry):
                cv = centers_v[pl.ds(i * _D + dd, 16)]
                return carry + cv[0] * ct_v[pl.ds(dd * 16, 16)]

            dots = lax.fori_loop(0, _D, _dot, zero16)
            d2 = jnp.maximum(norm2[i - 1] + norm2 - 2.0 * dots, 0.0)
            hinge_p = jnp.maximum(_TWO_DELTA_D - _sqrt16(d2), 0.0)
            maskf = jnp.where(ids16 > i, 1.0, 0.0)
            pairsum_v = pairsum_v + _hsum(hinge_p * presentf * maskf) * presentf[i - 1]

        npairs_v = n_inst_v * (n_inst_v - 1.0) * 0.5
        dist_term_v = pairsum_v / jnp.maximum(npairs_v, 1.0)

        obuf[...] = (var_term_v + dist_term_v + _GAMMA * reg_term_v) * lane0
        pltpu.sync_copy(obuf, out_hbm.at[batch])


def kernel(embeddings, labels):
    emb2 = embeddings.reshape(_B, _N * _D)
    out = _loss_kernel(emb2, labels)
    return jnp.sum(out[:, 0]) / jnp.float32(_B)


# speedup vs baseline: 1.4632x; 1.4632x over previous
import functools

import jax
import jax.numpy as jnp
from jax import lax
from jax.experimental import pallas as pl
from jax.experimental.pallas import tpu as pltpu
from jax.experimental.pallas import tpu_sc as plsc

_B = 8
_N = 4096
_D = 64
_NID = 32
_TPB = 4
_NPT = _N // _TPB
_DELTA_V = 0.5
_TWO_DELTA_D = 3.0
_GAMMA = 0.001


def _sqrt16(v):
    i = plsc.bitcast(v, jnp.int32)
    g = plsc.bitcast(lax.shift_right_logical(i, 1) + 0x1FBD1DF5, jnp.float32)
    for _ in range(3):
        g = 0.5 * (g + v / g)
    return g


_mesh = plsc.VectorSubcoreMesh(
    core_axis_name="c", subcore_axis_name="s", num_cores=2, num_subcores=16
)


@functools.partial(
    pl.kernel,
    out_type=jax.ShapeDtypeStruct((_B, 16), jnp.float32),
    mesh=_mesh,
    compiler_params=pltpu.CompilerParams(needs_layout_passes=False),
    scratch_types=[
        pltpu.VMEM((_NPT * _D,), jnp.float32),
        pltpu.VMEM((_NPT,), jnp.int32),
        pltpu.VMEM((_NID * _D,), jnp.float32),
        pltpu.VMEM((_NID * _D,), jnp.float32),
        pltpu.VMEM((_NID * 16,), jnp.float32),
        pltpu.VMEM((_NID * 16,), jnp.float32),
        pltpu.VMEM((_NID * 16,), jnp.float32),
        pltpu.VMEM((16 * 16,), jnp.float32),
        pltpu.VMEM((_D * 16,), jnp.float32),
        pltpu.VMEM((16,), jnp.float32),
        pltpu.VMEM_SHARED((16, _NID * _D), jnp.float32),
        pltpu.VMEM_SHARED((16, _NID * 16), jnp.float32),
        pltpu.VMEM_SHARED((16, _NID * 16), jnp.float32),
    ],
)
def _loss_kernel(
    emb_hbm, lab_hbm, out_hbm,
    emb_v, lab_v, sums_v, centers_v, counts_tab, hinge_tab, rbuf,
    qtab, ct_v, obuf, sums_sh, counts_sh, hinge_sh,
):
    c = lax.axis_index("c")
    s = lax.axis_index("s")
    batch = c * 4 + lax.shift_right_logical(s, 2)
    qtr = lax.bitwise_and(s, _TPB - 1)
    s0 = s - qtr
    zero16 = jnp.zeros((16,), jnp.float32)
    iota16 = lax.iota(jnp.int32, 16)
    ids16 = iota16 + 1
    lane0 = jnp.where(iota16 == 0, jnp.float32(1.0), jnp.float32(0.0))
    _onehots = [
        jnp.where(iota16 == k, jnp.float32(1.0), jnp.float32(0.0))
        for k in range(16)
    ]

    def _splat(x):
        return jnp.broadcast_to(x, (16,))

    def _rowsums(tab, row0):
        acc = zero16
        for i in range(16):
            acc = acc + _onehots[i] * jnp.sum(tab[pl.ds((row0 + i) * 16, 16)])
        return acc

    tile = batch * _TPB + qtr
    pltpu.sync_copy(emb_hbm.at[pl.ds(tile * (_NPT * _D), _NPT * _D)], emb_v)
    pltpu.sync_copy(lab_hbm.at[pl.ds(tile * _NPT, _NPT)], lab_v)

    def _zb(i, carry):
        sums_v[pl.ds(i * 16, 16)] = zero16
        return carry

    lax.fori_loop(0, (_NID * _D) // 16, _zb, 0)

    def _zt(i, carry):
        counts_tab[pl.ds(i * 16, 16)] = zero16
        hinge_tab[pl.ds(i * 16, 16)] = zero16
        return carry

    lax.fori_loop(0, _NID, _zt, 0)

    @plsc.parallel_loop(0, _NPT // 16, unroll=4)
    def _p1(g):
        lvec = lab_v[pl.ds(g * 16, 16)]
        base = g * 16
        for k in range(16):
            l = lvec[k]
            co = l * _D
            eo = (base + k) * _D
            for kk in range(_D // 16):
                plsc.addupdate(
                    sums_v.at[pl.ds(co + kk * 16, 16)],
                    emb_v[pl.ds(eo + kk * 16, 16)],
                )
            plsc.addupdate(counts_tab.at[pl.ds(l * 16, 16)], lane0)

    pltpu.sync_copy(sums_v, sums_sh.at[s])
    pltpu.sync_copy(counts_tab, counts_sh.at[s])
    plsc.subcore_barrier()

    for j in range(1, _TPB):
        slot = s0 + lax.bitwise_and(qtr + j, _TPB - 1)
        pltpu.sync_copy(counts_sh.at[slot], rbuf)

        def _accc(i, carry):
            sl = pl.ds(i * 16, 16)
            plsc.addupdate(counts_tab.at[sl], rbuf[sl])
            return carry

        lax.fori_loop(0, _NID, _accc, 0)
        pltpu.sync_copy(sums_sh.at[slot], centers_v)

        def _accs(i, carry):
            sl = pl.ds(i * 16, 16)
            plsc.addupdate(sums_v.at[sl], centers_v[sl])
            return carry

        lax.fori_loop(0, (_NID * _D) // 16, _accs, 0)

    rv0 = 1.0 / jnp.maximum(_rowsums(counts_tab, 0), 1.0)
    rv1 = 1.0 / jnp.maximum(_rowsums(counts_tab, 16), 1.0)
    for i in range(_NID):
        r = rv0[i] if i < 16 else rv1[i - 16]
        for kk in range(_D // 16):
            sl = pl.ds(i * _D + kk * 16, 16)
            centers_v[sl] = sums_v[sl] * r

    @plsc.parallel_loop(0, _NPT // 16, unroll=4)
    def _p2(g):
        base = g * 16
        lvec = lab_v[pl.ds(base, 16)]
        qv = zero16
        for k in range(16):
            l = lvec[k]
            co = l * _D
            eo = (base + k) * _D
            acc0 = None
            acc1 = None
            for kk in range(_D // 16):
                e = emb_v[pl.ds(eo + kk * 16, 16)]
                cc = centers_v[pl.ds(co + kk * 16, 16)]
                d = e - cc
                t = d * d
                if kk % 2 == 0:
                    acc0 = t if acc0 is None else acc0 + t
                else:
                    acc1 = t if acc1 is None else acc1 + t
            qv = qv + _onehots[k] * jnp.sum(acc0 + acc1)
        hv = jnp.maximum(_sqrt16(qv) - _DELTA_V, 0.0)
        for k in range(16):
            l = lvec[k]
            plsc.addupdate(hinge_tab.at[pl.ds(l * 16, 16)], hv * _onehots[k])

    pltpu.sync_copy(hinge_tab, hinge_sh.at[s])
    plsc.subcore_barrier()

    @pl.when(qtr == 0)
    def _lead_final():
        for j in range(1, _TPB):
            pltpu.sync_copy(hinge_sh.at[s + j], rbuf)

            def _acch(i, carry):
                sl = pl.ds(i * 16, 16)
                plsc.addupdate(hinge_tab.at[sl], rbuf[sl])
                return carry

            lax.fori_loop(0, _NID, _acch, 0)

        cnt = _rowsums(counts_tab, 1)
        hin = _rowsums(hinge_tab, 1)
        presentf = jnp.where(cnt > 0.0, 1.0, 0.0)
        n_inst_v = _splat(jnp.sum(presentf))
        safe_n_v = jnp.maximum(n_inst_v, 1.0)
        mean_d = hin / jnp.maximum(cnt, 1.0)
        var_term_v = _splat(jnp.sum(mean_d * presentf)) / safe_n_v

        def _tr(dd, carry):
            col = plsc.load_gather(centers_v, [ids16 * _D + dd])
            ct_v[pl.ds(dd * 16, 16)] = col
            return carry + col * col

        norm2 = lax.fori_loop(0, _D, _tr, zero16)
        reg_term_v = _splat(jnp.sum(_sqrt16(norm2) * presentf)) / safe_n_v

        pairsum_v = zero16
        for i in range(1, 17):
            def _dot(cc, carry):
                cv = centers_v[pl.ds(i * _D + cc * 16, 16)]
                acc = carry
                for t in range(16):
                    acc = acc + cv[t] * ct_v[pl.ds((cc * 16 + t) * 16, 16)]
                return acc

            dots = lax.fori_loop(0, _D // 16, _dot, zero16)
            d2 = jnp.maximum(norm2[i - 1] + norm2 - 2.0 * dots, 0.0)
            hinge_p = jnp.maximum(_TWO_DELTA_D - _sqrt16(d2), 0.0)
            maskf = jnp.where(ids16 > i, 1.0, 0.0)
            pairsum_v = pairsum_v + _splat(jnp.sum(hinge_p * presentf * maskf)) * presentf[i - 1]

        npairs_v = n_inst_v * (n_inst_v - 1.0) * 0.5
        dist_term_v = pairsum_v / jnp.maximum(npairs_v, 1.0)

        obuf[...] = (var_term_v + dist_term_v + _GAMMA * reg_term_v) * lane0
        pltpu.sync_copy(obuf, out_hbm.at[batch])


def kernel(embeddings, labels):
    emb_flat = embeddings.reshape(_B * _N * _D)
    lab_flat = labels.reshape(_B * _N)
    out = _loss_kernel(emb_flat, lab_flat)
    return jnp.sum(out[:, 0]) / jnp.float32(_B)


# speedup vs baseline: 1.4783x; 1.0104x over previous
import functools

import jax
import jax.numpy as jnp
from jax import lax
from jax.experimental import pallas as pl
from jax.experimental.pallas import tpu as pltpu
from jax.experimental.pallas import tpu_sc as plsc

_B = 8
_N = 4096
_D = 64
_NID = 32
_TPB = 4
_NPT = _N // _TPB
_DELTA_V = 0.5
_TWO_DELTA_D = 3.0
_GAMMA = 0.001


def _sqrt16(v):
    i = plsc.bitcast(v, jnp.int32)
    g = plsc.bitcast(lax.shift_right_logical(i, 1) + 0x1FBD1DF5, jnp.float32)
    for _ in range(3):
        g = 0.5 * (g + v / g)
    return g


_mesh = plsc.VectorSubcoreMesh(
    core_axis_name="c", subcore_axis_name="s", num_cores=2, num_subcores=16
)


@functools.partial(
    pl.kernel,
    out_type=jax.ShapeDtypeStruct((_B, 16), jnp.float32),
    mesh=_mesh,
    compiler_params=pltpu.CompilerParams(needs_layout_passes=False),
    scratch_types=[
        pltpu.VMEM((_NPT * _D // 128, 128), jnp.float32),
        pltpu.VMEM((_NPT // 128, 128), jnp.int32),
        pltpu.VMEM((_NID * _D,), jnp.float32),
        pltpu.VMEM((_NID * _D,), jnp.float32),
        pltpu.VMEM((_NID * 16,), jnp.float32),
        pltpu.VMEM((_NID * 16,), jnp.float32),
        pltpu.VMEM((_NID * 16,), jnp.float32),
        pltpu.VMEM((16 * 16,), jnp.float32),
        pltpu.VMEM((_D * 16,), jnp.float32),
        pltpu.VMEM((16,), jnp.float32),
        pltpu.VMEM_SHARED((16, _NID * _D), jnp.float32),
        pltpu.VMEM_SHARED((16, _NID * 16), jnp.float32),
        pltpu.VMEM_SHARED((16, _NID * 16), jnp.float32),
    ],
)
def _loss_kernel(
    emb_hbm, lab_hbm, out_hbm,
    emb_v, lab_v, sums_v, centers_v, counts_tab, hinge_tab, rbuf,
    qtab, ct_v, obuf, sums_sh, counts_sh, hinge_sh,
):
    c = lax.axis_index("c")
    s = lax.axis_index("s")
    batch = c * 4 + lax.shift_right_logical(s, 2)
    qtr = lax.bitwise_and(s, _TPB - 1)
    s0 = s - qtr
    zero16 = jnp.zeros((16,), jnp.float32)
    iota16 = lax.iota(jnp.int32, 16)
    ids16 = iota16 + 1
    lane0 = jnp.where(iota16 == 0, jnp.float32(1.0), jnp.float32(0.0))
    _onehots = [
        jnp.where(iota16 == k, jnp.float32(1.0), jnp.float32(0.0))
        for k in range(16)
    ]

    def _splat(x):
        return jnp.broadcast_to(x, (16,))

    def _rowsums(tab, row0):
        acc = zero16
        for i in range(16):
            acc = acc + _onehots[i] * jnp.sum(tab[pl.ds((row0 + i) * 16, 16)])
        return acc

    pltpu.sync_copy(
        emb_hbm.at[batch, pl.ds(qtr * (_NPT * _D // 128), _NPT * _D // 128)], emb_v
    )
    pltpu.sync_copy(
        lab_hbm.at[batch, pl.ds(qtr * (_NPT // 128), _NPT // 128)], lab_v
    )

    def _zb(i, carry):
        sums_v[pl.ds(i * 16, 16)] = zero16
        return carry

    lax.fori_loop(0, (_NID * _D) // 16, _zb, 0)

    def _zt(i, carry):
        counts_tab[pl.ds(i * 16, 16)] = zero16
        hinge_tab[pl.ds(i * 16, 16)] = zero16
        return carry

    lax.fori_loop(0, _NID, _zt, 0)

    @plsc.parallel_loop(0, _NPT // 16, unroll=4)
    def _p1(g):
        lvec = lab_v[lax.shift_right_logical(g, 3), pl.ds(lax.bitwise_and(g, 7) * 16, 16)]
        erow = g * 8
        for k in range(16):
            l = lvec[k]
            co = l * _D
            for kk in range(_D // 16):
                off = k * _D + kk * 16
                plsc.addupdate(
                    sums_v.at[pl.ds(co + kk * 16, 16)],
                    emb_v[erow + (off >> 7), pl.ds(off & 127, 16)],
                )
            plsc.addupdate(counts_tab.at[pl.ds(l * 16, 16)], lane0)

    pltpu.sync_copy(sums_v, sums_sh.at[s])
    pltpu.sync_copy(counts_tab, counts_sh.at[s])
    plsc.subcore_barrier()

    for j in range(1, _TPB):
        slot = s0 + lax.bitwise_and(qtr + j, _TPB - 1)
        pltpu.sync_copy(counts_sh.at[slot], rbuf)

        def _accc(i, carry):
            sl = pl.ds(i * 16, 16)
            plsc.addupdate(counts_tab.at[sl], rbuf[sl])
            return carry

        lax.fori_loop(0, _NID, _accc, 0)
        pltpu.sync_copy(sums_sh.at[slot], centers_v)

        def _accs(i, carry):
            sl = pl.ds(i * 16, 16)
            plsc.addupdate(sums_v.at[sl], centers_v[sl])
            return carry

        lax.fori_loop(0, (_NID * _D) // 16, _accs, 0)

    rv0 = 1.0 / jnp.maximum(_rowsums(counts_tab, 0), 1.0)
    rv1 = 1.0 / jnp.maximum(_rowsums(counts_tab, 16), 1.0)
    for i in range(_NID):
        r = rv0[i] if i < 16 else rv1[i - 16]
        for kk in range(_D // 16):
            sl = pl.ds(i * _D + kk * 16, 16)
            centers_v[sl] = sums_v[sl] * r

    @plsc.parallel_loop(0, _NPT // 16, unroll=4)
    def _p2(g):
        lvec = lab_v[lax.shift_right_logical(g, 3), pl.ds(lax.bitwise_and(g, 7) * 16, 16)]
        erow = g * 8
        qv = zero16
        for k in range(16):
            l = lvec[k]
            co = l * _D
            acc0 = None
            acc1 = None
            for kk in range(_D // 16):
                off = k * _D + kk * 16
                e = emb_v[erow + (off >> 7), pl.ds(off & 127, 16)]
                cc = centers_v[pl.ds(co + kk * 16, 16)]
                d = e - cc
                t = d * d
                if kk % 2 == 0:
                    acc0 = t if acc0 is None else acc0 + t
                else:
                    acc1 = t if acc1 is None else acc1 + t
            qv = qv + _onehots[k] * jnp.sum(acc0 + acc1)
        hv = jnp.maximum(_sqrt16(qv) - _DELTA_V, 0.0)
        for k in range(16):
            l = lvec[k]
            plsc.addupdate(hinge_tab.at[pl.ds(l * 16, 16)], hv * _onehots[k])

    pltpu.sync_copy(hinge_tab, hinge_sh.at[s])
    plsc.subcore_barrier()

    @pl.when(qtr == 0)
    def _lead_final():
        for j in range(1, _TPB):
            pltpu.sync_copy(hinge_sh.at[s + j], rbuf)

            def _acch(i, carry):
                sl = pl.ds(i * 16, 16)
                plsc.addupdate(hinge_tab.at[sl], rbuf[sl])
                return carry

            lax.fori_loop(0, _NID, _acch, 0)

        cnt = _rowsums(counts_tab, 1)
        hin = _rowsums(hinge_tab, 1)
        presentf = jnp.where(cnt > 0.0, 1.0, 0.0)
        n_inst_v = _splat(jnp.sum(presentf))
        safe_n_v = jnp.maximum(n_inst_v, 1.0)
        mean_d = hin / jnp.maximum(cnt, 1.0)
        var_term_v = _splat(jnp.sum(mean_d * presentf)) / safe_n_v

        def _tr(dd, carry):
            col = plsc.load_gather(centers_v, [ids16 * _D + dd])
            ct_v[pl.ds(dd * 16, 16)] = col
            return carry + col * col

        norm2 = lax.fori_loop(0, _D, _tr, zero16)
        reg_term_v = _splat(jnp.sum(_sqrt16(norm2) * presentf)) / safe_n_v

        pairsum_v = zero16
        for i in range(1, 17):
            def _dot(cc, carry):
                cv = centers_v[pl.ds(i * _D + cc * 16, 16)]
                acc = carry
                for t in range(16):
                    acc = acc + cv[t] * ct_v[pl.ds((cc * 16 + t) * 16, 16)]
                return acc

            dots = lax.fori_loop(0, _D // 16, _dot, zero16)
            d2 = jnp.maximum(norm2[i - 1] + norm2 - 2.0 * dots, 0.0)
            hinge_p = jnp.maximum(_TWO_DELTA_D - _sqrt16(d2), 0.0)
            maskf = jnp.where(ids16 > i, 1.0, 0.0)
            pairsum_v = pairsum_v + _splat(jnp.sum(hinge_p * presentf * maskf)) * presentf[i - 1]

        npairs_v = n_inst_v * (n_inst_v - 1.0) * 0.5
        dist_term_v = pairsum_v / jnp.maximum(npairs_v, 1.0)

        obuf[...] = (var_term_v + dist_term_v + _GAMMA * reg_term_v) * lane0
        pltpu.sync_copy(obuf, out_hbm.at[batch])


def kernel(embeddings, labels):
    emb3 = embeddings.reshape(_B, _N * _D // 128, 128)
    lab3 = labels.reshape(_B, _N // 128, 128)
    out = _loss_kernel(emb3, lab3)
    return jnp.sum(out[:, 0]) / jnp.float32(_B)
